# Initial kernel scaffold; baseline (speedup 1.0000x reference)
import jax
import jax.numpy as jnp
from jax.experimental import pallas as pl


def kernel(x, edge_index, W0, b0, g0, be0, W1, b1, g1, be1, W2, b2, g2, be2, W3, b3, g3, be3, Wo, bo):
    raise NotImplementedError("write your pallas kernel here")


# speedup vs baseline: 2.9873x; 2.9873x over previous
import functools

import jax
import jax.numpy as jnp
from jax import lax
from jax.experimental import pallas as pl
from jax.experimental.pallas import tpu as pltpu
from jax.experimental.pallas import tpu_sc as plsc

N = 10000
D = 256
DH = 128
E = 160000
NCORE = 2
NSUB = 16
EB = 128
NB = 80
EPS = NB * EB
EPAD = NSUB * EPS
TROWS = 10112
ZR = TROWS // NSUB
BN = 1000


_INV_SQRT2 = 0.7071067811865476


def _gelu(h):
    return 0.5 * h * (1.0 + lax.erf(h * _INV_SQRT2))


def _mm0_body(x_ref, w_ref, b_ref, o_ref):
    r = jnp.dot(x_ref[...], w_ref[...], preferred_element_type=jnp.float32)
    r = r + b_ref[...]
    o_ref[0] = r[:, :DH]
    o_ref[1] = r[:, DH:]


def _fused_body(a_ref, g_ref, be_ref, w_ref, b_ref, o_ref):
    a = jnp.concatenate([a_ref[0], a_ref[1]], axis=-1)
    mu = jnp.mean(a, axis=-1, keepdims=True)
    var = jnp.mean((a - mu) ** 2, axis=-1, keepdims=True)
    h = (a - mu) * lax.rsqrt(var + 1e-5) * g_ref[...] + be_ref[...]
    h = _gelu(h)
    r = jnp.dot(h, w_ref[...], preferred_element_type=jnp.float32)
    r = r + b_ref[...]
    o_ref[0] = r[:, :DH]
    o_ref[1] = r[:, DH:]


def _final_body(a_ref, g_ref, be_ref, w_ref, b_ref, o_ref):
    a = jnp.concatenate([a_ref[0], a_ref[1]], axis=-1)
    mu = jnp.mean(a, axis=-1, keepdims=True)
    var = jnp.mean((a - mu) ** 2, axis=-1, keepdims=True)
    h = (a - mu) * lax.rsqrt(var + 1e-5) * g_ref[...] + be_ref[...]
    h = _gelu(h)
    r = jnp.dot(h, w_ref[...], preferred_element_type=jnp.float32)
    o_ref[...] = r + b_ref[...]


def _mm0(x, W, b):
    return pl.pallas_call(
        _mm0_body,
        grid=(N // BN,),
        in_specs=[
            pl.BlockSpec((BN, D), lambda i: (i, 0)),
            pl.BlockSpec((D, D), lambda i: (0, 0)),
            pl.BlockSpec((1, D), lambda i: (0, 0)),
        ],
        out_specs=pl.BlockSpec((2, BN, DH), lambda i: (0, i, 0)),
        out_shape=jax.ShapeDtypeStruct((2, N, DH), jnp.float32),
    )(x, W, b.reshape(1, D))


def _fused(a, g, be, W, b):
    return pl.pallas_call(
        _fused_body,
        grid=(N // BN,),
        in_specs=[
            pl.BlockSpec((2, BN, DH), lambda i: (0, i, 0)),
            pl.BlockSpec((1, D), lambda i: (0, 0)),
            pl.BlockSpec((1, D), lambda i: (0, 0)),
            pl.BlockSpec((D, D), lambda i: (0, 0)),
            pl.BlockSpec((1, D), lambda i: (0, 0)),
        ],
        out_specs=pl.BlockSpec((2, BN, DH), lambda i: (0, i, 0)),
        out_shape=jax.ShapeDtypeStruct((2, N, DH), jnp.float32),
    )(a, g.reshape(1, D), be.reshape(1, D), W, b.reshape(1, D))


def _final(a, g, be, W, b):
    return pl.pallas_call(
        _final_body,
        grid=(N // BN,),
        in_specs=[
            pl.BlockSpec((2, BN, DH), lambda i: (0, i, 0)),
            pl.BlockSpec((1, D), lambda i: (0, 0)),
            pl.BlockSpec((1, D), lambda i: (0, 0)),
            pl.BlockSpec((D, D), lambda i: (0, 0)),
            pl.BlockSpec((1, D), lambda i: (0, 0)),
        ],
        out_specs=pl.BlockSpec((BN, D), lambda i: (i, 0)),
        out_shape=jax.ShapeDtypeStruct((N, D), jnp.float32),
    )(a, g.reshape(1, D), be.reshape(1, D), W, b.reshape(1, D))


_SC_MESH = plsc.VectorSubcoreMesh(
    core_axis_name="c", subcore_axis_name="s", num_cores=NCORE, num_subcores=NSUB
)


@functools.partial(
    pl.kernel,
    out_type=jax.ShapeDtypeStruct((NCORE, TROWS, DH), jnp.float32),
    mesh=_SC_MESH,
    scratch_types=[
        pltpu.VMEM_SHARED((TROWS, DH), jnp.float32),
        pltpu.VMEM((NB, EB), jnp.int32),
        pltpu.VMEM((NB, EB), jnp.int32),
        pltpu.VMEM((EB, DH), jnp.float32),
    ],
)
def _sc_scatter(hlin_hbm, srcg_hbm, dstg_hbm, zeros_hbm, out_hbm,
                table, src_v, dst_v, rows_v):
    cid = lax.axis_index("c")
    sid = lax.axis_index("s")

    pltpu.sync_copy(srcg_hbm.at[sid], src_v)
    pltpu.sync_copy(dstg_hbm.at[sid], dst_v)
    pltpu.sync_copy(zeros_hbm.at[pl.ds(sid * ZR, ZR)],
                    table.at[pl.ds(sid * ZR, ZR)])
    plsc.subcore_barrier()

    def body(j, carry):
        pltpu.sync_copy(hlin_hbm.at[cid].at[src_v.at[j]], rows_v)
        pltpu.sync_copy(rows_v, table.at[dst_v.at[j]], add=True)
        return carry

    lax.fori_loop(0, NB, body, 0)
    plsc.subcore_barrier()

    pltpu.sync_copy(table.at[pl.ds(sid * ZR, ZR)],
                    out_hbm.at[cid].at[pl.ds(sid * ZR, ZR)])


def kernel(x, edge_index, W0, b0, g0, be0, W1, b1, g1, be1,
           W2, b2, g2, be2, W3, b3, g3, be3, Wo, bo):
    src = edge_index[0]
    dst = edge_index[1]
    pad = EPAD - E
    srcg = jnp.concatenate(
        [src, jnp.zeros((pad,), jnp.int32)]).reshape(NSUB, NB, EB)
    dstg = jnp.concatenate(
        [dst, jnp.full((pad,), N, jnp.int32)]).reshape(NSUB, NB, EB)
    zeros = jnp.zeros((TROWS, DH), jnp.float32)

    hlin = _mm0(x, W0, b0)
    layers = [(g0, be0, W1, b1), (g1, be1, W2, b2), (g2, be2, W3, b3)]
    for (g, be, W, b) in layers:
        agg = _sc_scatter(hlin, srcg, dstg, zeros)
        hlin = _fused(agg, g, be, W, b)
    agg = _sc_scatter(hlin, srcg, dstg, zeros)
    return _final(agg, g3, be3, Wo, bo)


# speedup vs baseline: 3.6675x; 1.2277x over previous
import functools

import jax
import jax.numpy as jnp
from jax import lax
from jax.experimental import pallas as pl
from jax.experimental.pallas import tpu as pltpu
from jax.experimental.pallas import tpu_sc as plsc

N = 10000
D = 256
DH = 128
E = 160000
NCORE = 2
NSUB = 16
EB = 128
NB = 80
HNB = 40
EPS = NB * EB
EPAD = NSUB * EPS
TROWS = 10112
ZR = TROWS // NSUB
BN = 1000


_INV_SQRT2 = 0.7071067811865476


def _gelu(h):
    return 0.5 * h * (1.0 + lax.erf(h * _INV_SQRT2))


def _mm0_body(x_ref, w_ref, b_ref, o_ref):
    r = jnp.dot(x_ref[...], w_ref[...], preferred_element_type=jnp.float32)
    r = r + b_ref[...]
    o_ref[0] = r[:, :DH]
    o_ref[1] = r[:, DH:]


def _fused_body(a_ref, g_ref, be_ref, w_ref, b_ref, o_ref):
    a = jnp.concatenate([a_ref[0], a_ref[1]], axis=-1)
    mu = jnp.mean(a, axis=-1, keepdims=True)
    var = jnp.mean((a - mu) ** 2, axis=-1, keepdims=True)
    h = (a - mu) * lax.rsqrt(var + 1e-5) * g_ref[...] + be_ref[...]
    h = _gelu(h)
    r = jnp.dot(h, w_ref[...], preferred_element_type=jnp.float32)
    r = r + b_ref[...]
    o_ref[0] = r[:, :DH]
    o_ref[1] = r[:, DH:]


def _final_body(a_ref, g_ref, be_ref, w_ref, b_ref, o_ref):
    a = jnp.concatenate([a_ref[0], a_ref[1]], axis=-1)
    mu = jnp.mean(a, axis=-1, keepdims=True)
    var = jnp.mean((a - mu) ** 2, axis=-1, keepdims=True)
    h = (a - mu) * lax.rsqrt(var + 1e-5) * g_ref[...] + be_ref[...]
    h = _gelu(h)
    r = jnp.dot(h, w_ref[...], preferred_element_type=jnp.float32)
    o_ref[...] = r + b_ref[...]


def _mm0(x, W, b):
    return pl.pallas_call(
        _mm0_body,
        grid=(N // BN,),
        in_specs=[
            pl.BlockSpec((BN, D), lambda i: (i, 0)),
            pl.BlockSpec((D, D), lambda i: (0, 0)),
            pl.BlockSpec((1, D), lambda i: (0, 0)),
        ],
        out_specs=pl.BlockSpec((2, BN, DH), lambda i: (0, i, 0)),
        out_shape=jax.ShapeDtypeStruct((2, N, DH), jnp.float32),
    )(x, W, b.reshape(1, D))


def _fused(a, g, be, W, b):
    return pl.pallas_call(
        _fused_body,
        grid=(N // BN,),
        in_specs=[
            pl.BlockSpec((2, BN, DH), lambda i: (0, i, 0)),
            pl.BlockSpec((1, D), lambda i: (0, 0)),
            pl.BlockSpec((1, D), lambda i: (0, 0)),
            pl.BlockSpec((D, D), lambda i: (0, 0)),
            pl.BlockSpec((1, D), lambda i: (0, 0)),
        ],
        out_specs=pl.BlockSpec((2, BN, DH), lambda i: (0, i, 0)),
        out_shape=jax.ShapeDtypeStruct((2, N, DH), jnp.float32),
    )(a, g.reshape(1, D), be.reshape(1, D), W, b.reshape(1, D))


def _final(a, g, be, W, b):
    return pl.pallas_call(
        _final_body,
        grid=(N // BN,),
        in_specs=[
            pl.BlockSpec((2, BN, DH), lambda i: (0, i, 0)),
            pl.BlockSpec((1, D), lambda i: (0, 0)),
            pl.BlockSpec((1, D), lambda i: (0, 0)),
            pl.BlockSpec((D, D), lambda i: (0, 0)),
            pl.BlockSpec((1, D), lambda i: (0, 0)),
        ],
        out_specs=pl.BlockSpec((BN, D), lambda i: (i, 0)),
        out_shape=jax.ShapeDtypeStruct((N, D), jnp.float32),
    )(a, g.reshape(1, D), be.reshape(1, D), W, b.reshape(1, D))


_SC_MESH = plsc.VectorSubcoreMesh(
    core_axis_name="c", subcore_axis_name="s", num_cores=NCORE, num_subcores=NSUB
)


@functools.partial(
    pl.kernel,
    out_type=jax.ShapeDtypeStruct((NCORE, TROWS, DH), jnp.float32),
    mesh=_SC_MESH,
    scratch_types=[
        pltpu.VMEM_SHARED((TROWS, DH), jnp.float32),
        pltpu.VMEM((HNB, EB), jnp.int32),
        pltpu.VMEM((HNB, EB), jnp.int32),
        pltpu.VMEM((2, EB, DH), jnp.float32),
        pltpu.SemaphoreType.DMA,
    ],
)
def _sc_scatter(hlin_hbm, srcg_hbm, dstg_hbm, zeros_hbm, out_hbm,
                table, src_v, dst_v, rows_v, gsem):
    cid = lax.axis_index("c")
    sid = lax.axis_index("s")

    pltpu.sync_copy(zeros_hbm.at[pl.ds(sid * ZR, ZR)],
                    table.at[pl.ds(sid * ZR, ZR)])
    plsc.subcore_barrier()

    def body(j, carry):
        nxt = j + 1

        @pl.when(nxt < HNB)
        def _():
            pltpu.async_copy(hlin_hbm.at[cid].at[src_v.at[nxt]],
                             rows_v.at[nxt % 2], gsem)

        pltpu.make_async_copy(hlin_hbm.at[cid].at[src_v.at[j]],
                              rows_v.at[j % 2], gsem).wait()
        pltpu.sync_copy(rows_v.at[j % 2], table.at[dst_v.at[j]], add=True)
        return carry

    for h in range(NB // HNB):
        pltpu.sync_copy(srcg_hbm.at[sid].at[pl.ds(h * HNB, HNB)], src_v)
        pltpu.sync_copy(dstg_hbm.at[sid].at[pl.ds(h * HNB, HNB)], dst_v)
        pltpu.async_copy(hlin_hbm.at[cid].at[src_v.at[0]], rows_v.at[0], gsem)
        lax.fori_loop(0, HNB, body, 0)
    plsc.subcore_barrier()

    pltpu.sync_copy(table.at[pl.ds(sid * ZR, ZR)],
                    out_hbm.at[cid].at[pl.ds(sid * ZR, ZR)])


def kernel(x, edge_index, W0, b0, g0, be0, W1, b1, g1, be1,
           W2, b2, g2, be2, W3, b3, g3, be3, Wo, bo):
    src = edge_index[0]
    dst = edge_index[1]
    pad = EPAD - E
    srcg = jnp.concatenate(
        [src, jnp.zeros((pad,), jnp.int32)]).reshape(NSUB, NB, EB)
    dstg = jnp.concatenate(
        [dst, jnp.full((pad,), N, jnp.int32)]).reshape(NSUB, NB, EB)
    zeros = jnp.zeros((TROWS, DH), jnp.float32)

    hlin = _mm0(x, W0, b0)
    layers = [(g0, be0, W1, b1), (g1, be1, W2, b2), (g2, be2, W3, b3)]
    for (g, be, W, b) in layers:
        agg = _sc_scatter(hlin, srcg, dstg, zeros)
        hlin = _fused(agg, g, be, W, b)
    agg = _sc_scatter(hlin, srcg, dstg, zeros)
    return _final(agg, g3, be3, Wo, bo)
